# baseline (speedup 1.0000x reference)
import functools

import jax
import jax.numpy as jnp
from jax import lax
from jax.experimental import pallas as pl
from jax.experimental.pallas import tpu as pltpu
from jax.experimental.pallas import tpu_sc as plsc

EMBED_DIM = 64
BATCH = 4096
HIST = 200
N_ROWS = 1000000

TOTAL = BATCH * HIST
NUM_WORKERS = 32
PER_WORKER = TOTAL // NUM_WORKERS
CHUNK = 800
NUM_CHUNKS = PER_WORKER // CHUNK
NBUF = 2

REPACK_BLK = 512
REPACK_GRID = (N_ROWS + REPACK_BLK - 1) // REPACK_BLK

_mesh = plsc.VectorSubcoreMesh(core_axis_name="c", subcore_axis_name="s")


@functools.partial(
    pl.pallas_call,
    grid=(REPACK_GRID,),
    in_specs=[pl.BlockSpec((EMBED_DIM, REPACK_BLK), lambda i: (0, i))],
    out_specs=pl.BlockSpec((REPACK_BLK, 2 * EMBED_DIM), lambda i: (i, 0)),
    out_shape=jax.ShapeDtypeStruct((N_ROWS, 2 * EMBED_DIM), jnp.float32),
)
def _repack(tt_ref, out_ref):
    eye = (
        lax.broadcasted_iota(jnp.int32, (EMBED_DIM, EMBED_DIM), 0)
        == lax.broadcasted_iota(jnp.int32, (EMBED_DIM, EMBED_DIM), 1)
    ).astype(jnp.float32)
    t = lax.dot_general(
        tt_ref[...],
        eye,
        (((0,), (0,)), ((), ())),
        preferred_element_type=jnp.float32,
    )
    out_ref[...] = jnp.concatenate([t, t], axis=1)


@functools.partial(
    pl.kernel,
    mesh=_mesh,
    out_type=jax.ShapeDtypeStruct((TOTAL, 128), jnp.float32),
    scratch_types=[
        pltpu.VMEM((NUM_CHUNKS, CHUNK), jnp.int32),
        pltpu.VMEM((NBUF, CHUNK, EMBED_DIM), jnp.float32),
        pltpu.SemaphoreType.DMA((NBUF,)),
        pltpu.SemaphoreType.DMA((NBUF,)),
    ],
    compiler_params=pltpu.CompilerParams(use_tc_tiling_on_sc=False),
)
def _gather_kernel(idx_hbm, table_hbm, out_hbm, idx_v, rows_v, gsem, osem):
    wid = lax.axis_index("s") * 2 + lax.axis_index("c")
    base = wid * PER_WORKER

    pltpu.sync_copy(idx_hbm.at[pl.ds(wid * NUM_CHUNKS, NUM_CHUNKS)], idx_v)

    def fire_gather(slot, i):
        pltpu.async_copy(
            table_hbm.at[idx_v.at[i]],
            rows_v.at[slot],
            gsem.at[slot],
        )

    def wait_gather(slot, i):
        pltpu.make_async_copy(
            table_hbm.at[idx_v.at[i]],
            rows_v.at[slot],
            gsem.at[slot],
        ).wait()

    def fire_out(slot, i):
        off = pl.multiple_of(base + i * CHUNK, 8)
        pltpu.async_copy(
            rows_v.at[slot],
            out_hbm.at[pl.ds(off, CHUNK), pl.ds(0, EMBED_DIM)],
            osem.at[slot],
        )

    def wait_out(slot, i):
        off = pl.multiple_of(base + i * CHUNK, 8)
        pltpu.make_async_copy(
            rows_v.at[slot],
            out_hbm.at[pl.ds(off, CHUNK), pl.ds(0, EMBED_DIM)],
            osem.at[slot],
        ).wait()

    for b in range(NBUF):
        fire_gather(b, b)

    def body(g, _):
        for b in range(NBUF):
            i = g * NBUF + b
            wait_gather(b, i)
            fire_out(b, i)
            wait_out(b, i)
            fire_gather(b, i + NBUF)
        return 0

    lax.fori_loop(0, (NUM_CHUNKS - NBUF) // NBUF, body, 0)

    for b in range(NBUF):
        i = NUM_CHUNKS - NBUF + b
        wait_gather(b, i)
        fire_out(b, i)
    for b in range(NBUF):
        i = NUM_CHUNKS - NBUF + b
        wait_out(b, i)


def kernel(input_ids, table):
    flat_ids2 = (input_ids.reshape(NUM_WORKERS * NUM_CHUNKS, CHUNK) * 2).astype(
        jnp.int32
    )
    table2 = _repack(table.T).reshape(2 * N_ROWS, EMBED_DIM)
    out = _gather_kernel(flat_ids2, table2)
    return out[:, :EMBED_DIM].reshape(BATCH, HIST, EMBED_DIM)


# speedup vs baseline: 2.5181x; 2.5181x over previous
import functools

import jax
import jax.numpy as jnp
from jax import lax
from jax.experimental import pallas as pl
from jax.experimental.pallas import tpu as pltpu
from jax.experimental.pallas import tpu_sc as plsc

EMBED_DIM = 64
BATCH = 4096
HIST = 200
N_ROWS = 1000000

TOTAL = BATCH * HIST
NUM_WORKERS = 32
PER_WORKER = TOTAL // NUM_WORKERS
CHUNK = 800
NUM_CHUNKS = PER_WORKER // CHUNK
NBUF = 2

REPACK_BLK = 8192
REPACK_GRID = (N_ROWS + REPACK_BLK - 1) // REPACK_BLK

_mesh = plsc.VectorSubcoreMesh(core_axis_name="c", subcore_axis_name="s")


@functools.partial(
    pl.pallas_call,
    grid=(REPACK_GRID,),
    in_specs=[
        pl.BlockSpec((EMBED_DIM, REPACK_BLK), lambda i: (0, i)),
        pl.BlockSpec((EMBED_DIM, EMBED_DIM), lambda i: (0, 0)),
    ],
    out_specs=pl.BlockSpec((REPACK_BLK, 2 * EMBED_DIM), lambda i: (i, 0)),
    out_shape=jax.ShapeDtypeStruct((N_ROWS, 2 * EMBED_DIM), jnp.float32),
)
def _repack(tt_ref, eye_ref, out_ref):
    t = lax.dot_general(
        tt_ref[...],
        eye_ref[...],
        (((0,), (0,)), ((), ())),
        preferred_element_type=jnp.float32,
    )
    out_ref[...] = jnp.concatenate([t, t], axis=1)


@functools.partial(
    pl.kernel,
    mesh=_mesh,
    out_type=jax.ShapeDtypeStruct((TOTAL, 128), jnp.float32),
    scratch_types=[
        pltpu.VMEM((NUM_CHUNKS, CHUNK), jnp.int32),
        pltpu.VMEM((NBUF, CHUNK, EMBED_DIM), jnp.float32),
        pltpu.SemaphoreType.DMA((NBUF,)),
        pltpu.SemaphoreType.DMA((NBUF,)),
    ],
    compiler_params=pltpu.CompilerParams(use_tc_tiling_on_sc=False),
)
def _gather_kernel(idx_hbm, table_hbm, out_hbm, idx_v, rows_v, gsem, osem):
    wid = lax.axis_index("s") * 2 + lax.axis_index("c")
    base = wid * PER_WORKER

    pltpu.sync_copy(idx_hbm.at[pl.ds(wid * NUM_CHUNKS, NUM_CHUNKS)], idx_v)

    def fire_gather(slot, i):
        pltpu.async_copy(
            table_hbm.at[idx_v.at[i]],
            rows_v.at[slot],
            gsem.at[slot],
        )

    def wait_gather(slot, i):
        pltpu.make_async_copy(
            table_hbm.at[idx_v.at[i]],
            rows_v.at[slot],
            gsem.at[slot],
        ).wait()

    def fire_out(slot, i):
        off = pl.multiple_of(base + i * CHUNK, 8)
        pltpu.async_copy(
            rows_v.at[slot],
            out_hbm.at[pl.ds(off, CHUNK), pl.ds(0, EMBED_DIM)],
            osem.at[slot],
        )

    def wait_out(slot, i):
        off = pl.multiple_of(base + i * CHUNK, 8)
        pltpu.make_async_copy(
            rows_v.at[slot],
            out_hbm.at[pl.ds(off, CHUNK), pl.ds(0, EMBED_DIM)],
            osem.at[slot],
        ).wait()

    for b in range(NBUF):
        fire_gather(b, b)

    def body(g, _):
        for b in range(NBUF):
            i = g * NBUF + b
            wait_gather(b, i)
            fire_out(b, i)
            wait_out(b, i)
            fire_gather(b, i + NBUF)
        return 0

    lax.fori_loop(0, (NUM_CHUNKS - NBUF) // NBUF, body, 0)

    for b in range(NBUF):
        i = NUM_CHUNKS - NBUF + b
        wait_gather(b, i)
        fire_out(b, i)
    for b in range(NBUF):
        i = NUM_CHUNKS - NBUF + b
        wait_out(b, i)


def kernel(input_ids, table):
    flat_ids2 = (input_ids.reshape(NUM_WORKERS * NUM_CHUNKS, CHUNK) * 2).astype(
        jnp.int32
    )
    eye = jnp.eye(EMBED_DIM, dtype=jnp.float32)
    table2 = _repack(table.T, eye).reshape(2 * N_ROWS, EMBED_DIM)
    out = _gather_kernel(flat_ids2, table2)
    return out[:, :EMBED_DIM].reshape(BATCH, HIST, EMBED_DIM)


# speedup vs baseline: 2.6425x; 1.0494x over previous
import functools

import jax
import jax.numpy as jnp
from jax import lax
from jax.experimental import pallas as pl
from jax.experimental.pallas import tpu as pltpu
from jax.experimental.pallas import tpu_sc as plsc

EMBED_DIM = 64
BATCH = 4096
HIST = 200
N_ROWS = 1000000

TOTAL = BATCH * HIST
NUM_WORKERS = 32
PER_WORKER = TOTAL // NUM_WORKERS
CHUNK = 800
NUM_CHUNKS = PER_WORKER // CHUNK
NBUF = 2

REPACK_BLK = 16384
REPACK_GRID = (N_ROWS + REPACK_BLK - 1) // REPACK_BLK

_mesh = plsc.VectorSubcoreMesh(core_axis_name="c", subcore_axis_name="s")


@functools.partial(
    pl.pallas_call,
    grid=(REPACK_GRID,),
    in_specs=[
        pl.BlockSpec((EMBED_DIM, REPACK_BLK), lambda i: (0, i)),
        pl.BlockSpec((EMBED_DIM, EMBED_DIM), lambda i: (0, 0)),
    ],
    out_specs=pl.BlockSpec((REPACK_BLK, 2 * EMBED_DIM), lambda i: (i, 0)),
    out_shape=jax.ShapeDtypeStruct((N_ROWS, 2 * EMBED_DIM), jnp.float32),
)
def _repack(tt_ref, eye_ref, out_ref):
    t = lax.dot_general(
        tt_ref[...],
        eye_ref[...],
        (((0,), (0,)), ((), ())),
        preferred_element_type=jnp.float32,
    )
    out_ref[...] = jnp.concatenate([t, t], axis=1)


@functools.partial(
    pl.kernel,
    mesh=_mesh,
    out_type=jax.ShapeDtypeStruct((TOTAL, 128), jnp.float32),
    scratch_types=[
        pltpu.VMEM((NUM_CHUNKS, CHUNK), jnp.int32),
        pltpu.VMEM((NBUF, CHUNK, EMBED_DIM), jnp.float32),
        pltpu.SemaphoreType.DMA((NBUF,)),
        pltpu.SemaphoreType.DMA((NBUF,)),
    ],
    compiler_params=pltpu.CompilerParams(use_tc_tiling_on_sc=False),
)
def _gather_kernel(idx_hbm, table_hbm, out_hbm, idx_v, rows_v, gsem, osem):
    wid = lax.axis_index("s") * 2 + lax.axis_index("c")
    base = wid * PER_WORKER

    pltpu.sync_copy(idx_hbm.at[pl.ds(wid * NUM_CHUNKS, NUM_CHUNKS)], idx_v)

    def fire_gather(slot, i):
        pltpu.async_copy(
            table_hbm.at[idx_v.at[i]],
            rows_v.at[slot],
            gsem.at[slot],
        )

    def wait_gather(slot, i):
        pltpu.make_async_copy(
            table_hbm.at[idx_v.at[i]],
            rows_v.at[slot],
            gsem.at[slot],
        ).wait()

    def fire_out(slot, i):
        off = pl.multiple_of(base + i * CHUNK, 8)
        pltpu.async_copy(
            rows_v.at[slot],
            out_hbm.at[pl.ds(off, CHUNK), pl.ds(0, EMBED_DIM)],
            osem.at[slot],
        )

    def wait_out(slot, i):
        off = pl.multiple_of(base + i * CHUNK, 8)
        pltpu.make_async_copy(
            rows_v.at[slot],
            out_hbm.at[pl.ds(off, CHUNK), pl.ds(0, EMBED_DIM)],
            osem.at[slot],
        ).wait()

    for b in range(NBUF):
        fire_gather(b, b)

    def body(g, _):
        for b in range(NBUF):
            i = g * NBUF + b
            wait_gather(b, i)
            fire_out(b, i)
            wait_out(b, i)
            fire_gather(b, i + NBUF)
        return 0

    lax.fori_loop(0, (NUM_CHUNKS - NBUF) // NBUF, body, 0)

    for b in range(NBUF):
        i = NUM_CHUNKS - NBUF + b
        wait_gather(b, i)
        fire_out(b, i)
    for b in range(NBUF):
        i = NUM_CHUNKS - NBUF + b
        wait_out(b, i)


def kernel(input_ids, table):
    flat_ids2 = input_ids.reshape(NUM_WORKERS * NUM_CHUNKS, CHUNK) * 2
    eye = jnp.eye(EMBED_DIM, dtype=jnp.float32)
    table2 = _repack(table.T, eye).reshape(2 * N_ROWS, EMBED_DIM)
    out = _gather_kernel(flat_ids2, table2)
    return out[:, :EMBED_DIM].reshape(BATCH, HIST, EMBED_DIM)


# speedup vs baseline: 2.8614x; 1.0828x over previous
import functools

import jax
import jax.numpy as jnp
from jax import lax
from jax.experimental import pallas as pl
from jax.experimental.pallas import tpu as pltpu
from jax.experimental.pallas import tpu_sc as plsc

EMBED_DIM = 64
BATCH = 4096
HIST = 200
N_ROWS = 1000000

TOTAL = BATCH * HIST
NUM_WORKERS = 32
PER_WORKER = TOTAL // NUM_WORKERS
CHUNK = 800
NUM_CHUNKS = PER_WORKER // CHUNK
NBUF = 2

REPACK_BLK = 32768
REPACK_GRID = (N_ROWS + REPACK_BLK - 1) // REPACK_BLK

_mesh = plsc.VectorSubcoreMesh(core_axis_name="c", subcore_axis_name="s")


@functools.partial(
    pl.pallas_call,
    grid=(REPACK_GRID,),
    in_specs=[
        pl.BlockSpec((EMBED_DIM, REPACK_BLK), lambda i: (0, i)),
        pl.BlockSpec((EMBED_DIM, 2 * EMBED_DIM), lambda i: (0, 0)),
    ],
    out_specs=pl.BlockSpec((REPACK_BLK, 2 * EMBED_DIM), lambda i: (i, 0)),
    out_shape=jax.ShapeDtypeStruct((N_ROWS, 2 * EMBED_DIM), jnp.float32),
)
def _repack(tt_ref, eye_ref, out_ref):
    out_ref[...] = lax.dot_general(
        tt_ref[...],
        eye_ref[...],
        (((0,), (0,)), ((), ())),
        preferred_element_type=jnp.float32,
    )


@functools.partial(
    pl.kernel,
    mesh=_mesh,
    out_type=jax.ShapeDtypeStruct((TOTAL, 128), jnp.float32),
    scratch_types=[
        pltpu.VMEM((NUM_CHUNKS, CHUNK), jnp.int32),
        pltpu.VMEM((NBUF, CHUNK, EMBED_DIM), jnp.float32),
        pltpu.SemaphoreType.DMA((NBUF,)),
        pltpu.SemaphoreType.DMA((NBUF,)),
    ],
    compiler_params=pltpu.CompilerParams(use_tc_tiling_on_sc=False),
)
def _gather_kernel(idx_hbm, table_hbm, out_hbm, idx_v, rows_v, gsem, osem):
    wid = lax.axis_index("s") * 2 + lax.axis_index("c")
    base = wid * PER_WORKER

    pltpu.sync_copy(idx_hbm.at[pl.ds(wid * NUM_CHUNKS, NUM_CHUNKS)], idx_v)

    def fire_gather(slot, i):
        pltpu.async_copy(
            table_hbm.at[idx_v.at[i]],
            rows_v.at[slot],
            gsem.at[slot],
        )

    def wait_gather(slot, i):
        pltpu.make_async_copy(
            table_hbm.at[idx_v.at[i]],
            rows_v.at[slot],
            gsem.at[slot],
        ).wait()

    def fire_out(slot, i):
        off = pl.multiple_of(base + i * CHUNK, 8)
        pltpu.async_copy(
            rows_v.at[slot],
            out_hbm.at[pl.ds(off, CHUNK), pl.ds(0, EMBED_DIM)],
            osem.at[slot],
        )

    def wait_out(slot, i):
        off = pl.multiple_of(base + i * CHUNK, 8)
        pltpu.make_async_copy(
            rows_v.at[slot],
            out_hbm.at[pl.ds(off, CHUNK), pl.ds(0, EMBED_DIM)],
            osem.at[slot],
        ).wait()

    for b in range(NBUF):
        fire_gather(b, b)

    def body(g, _):
        for b in range(NBUF):
            i = g * NBUF + b
            wait_gather(b, i)
            fire_out(b, i)
            wait_out(b, i)
            fire_gather(b, i + NBUF)
        return 0

    lax.fori_loop(0, (NUM_CHUNKS - NBUF) // NBUF, body, 0)

    for b in range(NBUF):
        i = NUM_CHUNKS - NBUF + b
        wait_gather(b, i)
        fire_out(b, i)
    for b in range(NBUF):
        i = NUM_CHUNKS - NBUF + b
        wait_out(b, i)


def kernel(input_ids, table):
    flat_ids2 = input_ids.reshape(NUM_WORKERS * NUM_CHUNKS, CHUNK) * 2
    eye = jnp.eye(EMBED_DIM, dtype=jnp.float32)
    eye2 = jnp.concatenate([eye, eye], axis=1)
    table2 = _repack(table.T, eye2).reshape(2 * N_ROWS, EMBED_DIM)
    out = _gather_kernel(flat_ids2, table2)
    return out[:, :EMBED_DIM].reshape(BATCH, HIST, EMBED_DIM)
